# baseline (speedup 1.0000x reference)
import functools
import jax
import jax.numpy as jnp
from jax import lax
from jax.experimental import pallas as pl
from jax.experimental.pallas import tpu as pltpu
from jax.experimental.pallas import tpu_sc as plsc

_B, _H, _W, _C = 2, 64, 512, 64
_OH, _OW = 32, 256
_K = 16
_R2 = 2.5 * 2.5
_N = _OH * _OW
_M = _B * _N * _K
_NP = _H * _W
_ZROW = _B * _NP
_NT = _ZROW + 8
_PAIRS = _B * _OH
_NW = 32
_PPW = _PAIRS // _NW


def _sc_body(xyzflat, xyztab, ptstab, fpts, fxyz,
             stage, slots, pbuf, xbuf, sem, sem2):
    i32 = jnp.int32
    wid = lax.axis_index("s") * 2 + lax.axis_index("c")
    lane = lax.iota(i32, 16)

    def do_pair(k, _):
        pair = wid * _PPW + k
        b = pair // _OH
        oh = pair % _OH
        s = jnp.clip(2 * oh - 3, 0, _H - 6)
        pltpu.sync_copy(
            xyzflat.at[pl.ds((b * _NP + s * _W) * 4, 6 * _W * 4)], stage)

        def init(t, _):
            slots[t // 8, pl.ds((t % 8) * 16, 16)] = jnp.full((16,), _ZROW, i32)
            return _
        lax.fori_loop(0, 256, init, 0)

        def do_cg(cg, _):
            ow = cg * 16 + lane
            cidx = ((2 * oh - s) * _W + 2 * ow) * 4
            cx = plsc.load_gather(stage, [cidx])
            cy = plsc.load_gather(stage, [cidx + 1])
            cz = plsc.load_gather(stage, [cidx + 2])

            def do_cand(j, cnt):
                dh = j // 12 - 3
                dw = j % 12 - 6
                row = 2 * oh + dh
                rowok = jnp.logical_and(row >= 0, row < _H)
                rb = jnp.clip(row - s, 0, 5)
                col = 2 * ow + dw
                colok = jnp.logical_and(col >= 0, col < _W)
                colc = jnp.clip(col, 0, _W - 1)
                gi = (rb * _W + colc) * 4
                gx = plsc.load_gather(stage, [gi])
                gy = plsc.load_gather(stage, [gi + 1])
                gz = plsc.load_gather(stage, [gi + 2])
                d2 = ((gx - cx) * (gx - cx) + (gy - cy) * (gy - cy)
                      + (gz - cz) * (gz - cz))
                valid = jnp.logical_and(jnp.logical_and(colok, d2 < _R2),
                                        rowok)
                sel = jnp.logical_and(valid, cnt < _K)
                slot = ow * _K + cnt
                gidx = b * _NP + row * _W + colc
                plsc.store_scatter(slots, [slot // 128, slot % 128],
                                   gidx, mask=sel)
                return cnt + valid.astype(i32)

            lax.fori_loop(0, 72, do_cand, jnp.zeros((16,), i32))
            return _
        lax.fori_loop(0, 16, do_cg, 0)

        def do_chunk(ci, _):
            r0 = pair * (_OW * _K) + ci * 128
            cp = pltpu.async_copy(ptstab.at[slots.at[ci]], pbuf, sem)
            cx2 = pltpu.async_copy(xyztab.at[slots.at[ci]], xbuf, sem2)
            cp.wait()
            cx2.wait()
            pltpu.sync_copy(pbuf, fpts.at[pl.ds(r0, 128)])
            pltpu.sync_copy(xbuf, fxyz.at[pl.ds(r0, 128)])
            return _
        lax.fori_loop(0, 32, do_chunk, 0)
        return _

    lax.fori_loop(0, _PPW, do_pair, 0)


def _tc_body(fpts, fxyz, nxr, w1x, w1p, b1, w2, b2, out):
    f32 = jnp.float32
    pts = fpts[...]
    xd = fxyz[...][:, 0:3] - nxr[...]
    h1 = (jnp.dot(pts.astype(jnp.bfloat16), w1p[...].astype(jnp.bfloat16),
                  preferred_element_type=f32)
          + jnp.dot(xd, w1x[...], preferred_element_type=f32) + b1[...])
    h1 = jnp.maximum(h1, 0.0)
    h2 = jnp.dot(h1.astype(jnp.bfloat16), w2[...].astype(jnp.bfloat16),
                 preferred_element_type=f32) + b2[...]
    h2 = jnp.maximum(h2, 0.0)
    out[...] = jnp.max(h2.reshape(_OW, _K, 128), axis=1)


def kernel(xyz_proj, points_proj, xyz_sampled_proj, W1, b1, W2, b2):
    f32 = jnp.float32
    xyz4 = jnp.pad(xyz_proj.reshape(_B * _NP, 3), ((0, 8), (0, 13)))
    ptstab = jnp.pad(points_proj.reshape(_B * _NP, _C), ((0, 8), (0, 0)))
    xyzflat = jnp.pad(xyz_proj.reshape(_B * _NP, 3),
                      ((0, 0), (0, 1))).reshape(_B * _NP * 4)

    mesh = plsc.VectorSubcoreMesh(core_axis_name="c", subcore_axis_name="s")
    fpts, fxyz = pl.kernel(
        _sc_body,
        out_type=[jax.ShapeDtypeStruct((_M, _C), f32),
                  jax.ShapeDtypeStruct((_M, 16), f32)],
        mesh=mesh,
        scratch_types=[
            pltpu.VMEM((6 * _W * 4,), f32),
            pltpu.VMEM((32, 128), jnp.int32),
            pltpu.VMEM((128, _C), f32),
            pltpu.VMEM((128, 16), f32),
            pltpu.SemaphoreType.DMA,
            pltpu.SemaphoreType.DMA,
        ],
        compiler_params=pltpu.CompilerParams(needs_layout_passes=False,
                                             use_tc_tiling_on_sc=False),
    )(xyzflat, xyz4, ptstab)

    return _tc_part(fpts, fxyz, xyz_sampled_proj, W1, b1, W2, b2)


def _tc_part(fpts, fxyz, xyz_sampled_proj, W1, b1, W2, b2):
    f32 = jnp.float32
    nxr = jnp.broadcast_to(
        xyz_sampled_proj.reshape(_B, _N, 1, 3),
        (_B, _N, _K, 3)).reshape(_M, 3)
    grid = (_M // (_OW * _K),)
    out = pl.pallas_call(
        _tc_body,
        grid=grid,
        in_specs=[
            pl.BlockSpec((_OW * _K, _C), lambda m: (m, 0)),
            pl.BlockSpec((_OW * _K, 16), lambda m: (m, 0)),
            pl.BlockSpec((_OW * _K, 3), lambda m: (m, 0)),
            pl.BlockSpec((3, 64), lambda m: (0, 0)),
            pl.BlockSpec((64, 64), lambda m: (0, 0)),
            pl.BlockSpec((1, 64), lambda m: (0, 0)),
            pl.BlockSpec((64, 128), lambda m: (0, 0)),
            pl.BlockSpec((1, 128), lambda m: (0, 0)),
        ],
        out_specs=pl.BlockSpec((_OW, 128), lambda m: (m, 0)),
        out_shape=jax.ShapeDtypeStruct((_B * _N, 128), f32),
        compiler_params=pltpu.CompilerParams(
            dimension_semantics=("arbitrary",)),
    )(fpts, fxyz, nxr, W1[0:3], W1[3:], b1.reshape(1, 64), W2,
      b2.reshape(1, 128))
    return (out.reshape(_B, _N, 128), out.reshape(_B, _OH, _OW, 128))


# speedup vs baseline: 1.0100x; 1.0100x over previous
import functools
import jax
import jax.numpy as jnp
from jax import lax
from jax.experimental import pallas as pl
from jax.experimental.pallas import tpu as pltpu
from jax.experimental.pallas import tpu_sc as plsc

_B, _H, _W, _C = 2, 64, 512, 64
_OH, _OW = 32, 256
_K = 16
_R2 = 2.5 * 2.5
_N = _OH * _OW
_M = _B * _N * _K
_NP = _H * _W
_ZROW = _B * _NP
_NT = _ZROW + 8
_PAIRS = _B * _OH
_NW = 32
_PPW = _PAIRS // _NW


def _sc_body(xyzflat, xyztab, ptstab, fpts, fxyz,
             stage, slots, pbuf, xbuf, sem, sem2):
    i32 = jnp.int32
    wid = lax.axis_index("s") * 2 + lax.axis_index("c")
    lane = lax.iota(i32, 16)

    def do_pair(k, _):
        pair = wid * _PPW + k
        b = pair // _OH
        oh = pair % _OH
        s = jnp.clip(2 * oh - 3, 0, _H - 6)
        pltpu.sync_copy(
            xyzflat.at[pl.ds((b * _NP + s * _W) * 4, 6 * _W * 4)], stage)

        def init(t, _):
            slots[t // 64, pl.ds((t % 64) * 16, 16)] = (
                jnp.full((16,), _ZROW, i32))
            return _
        lax.fori_loop(0, 256, init, 0)

        def do_cg(cg, _):
            ow = cg * 16 + lane
            cidx = ((2 * oh - s) * _W + 2 * ow) * 4
            cx = plsc.load_gather(stage, [cidx])
            cy = plsc.load_gather(stage, [cidx + 1])
            cz = plsc.load_gather(stage, [cidx + 2])

            def do_cand(j, cnt):
                dh = j // 12 - 3
                dw = j % 12 - 6
                row = 2 * oh + dh
                rowok = jnp.logical_and(row >= 0, row < _H)
                rb = jnp.clip(row - s, 0, 5)
                col = 2 * ow + dw
                colok = jnp.logical_and(col >= 0, col < _W)
                colc = jnp.clip(col, 0, _W - 1)
                gi = (rb * _W + colc) * 4
                gx = plsc.load_gather(stage, [gi])
                gy = plsc.load_gather(stage, [gi + 1])
                gz = plsc.load_gather(stage, [gi + 2])
                d2 = ((gx - cx) * (gx - cx) + (gy - cy) * (gy - cy)
                      + (gz - cz) * (gz - cz))
                valid = jnp.logical_and(jnp.logical_and(colok, d2 < _R2),
                                        rowok)
                sel = jnp.logical_and(valid, cnt < _K)
                slot = ow * _K + cnt
                gidx = b * _NP + row * _W + colc
                plsc.store_scatter(slots, [slot // 1024, slot % 1024],
                                   gidx, mask=sel)
                return cnt + valid.astype(i32)

            lax.fori_loop(0, 72, do_cand, jnp.zeros((16,), i32))
            return _
        lax.fori_loop(0, 16, do_cg, 0)

        def do_chunk(ci, _):
            r0 = pair * (_OW * _K) + ci * 1024
            cp = pltpu.async_copy(ptstab.at[slots.at[ci]], pbuf, sem)
            cx2 = pltpu.async_copy(xyztab.at[slots.at[ci]], xbuf, sem2)
            cp.wait()
            cx2.wait()
            pltpu.sync_copy(pbuf, fpts.at[pl.ds(r0, 1024)])
            pltpu.sync_copy(xbuf, fxyz.at[pl.ds(r0, 1024)])
            return _
        lax.fori_loop(0, 4, do_chunk, 0)
        return _

    lax.fori_loop(0, _PPW, do_pair, 0)


def _tc_body(fpts, fxyz, nxr, w1x, w1p, b1, w2, b2, out):
    f32 = jnp.float32
    pts = fpts[...]
    xd = fxyz[...][:, 0:3] - nxr[...]
    h1 = (jnp.dot(pts.astype(jnp.bfloat16), w1p[...].astype(jnp.bfloat16),
                  preferred_element_type=f32)
          + jnp.dot(xd, w1x[...], preferred_element_type=f32) + b1[...])
    h1 = jnp.maximum(h1, 0.0)
    h2 = jnp.dot(h1.astype(jnp.bfloat16), w2[...].astype(jnp.bfloat16),
                 preferred_element_type=f32) + b2[...]
    h2 = jnp.maximum(h2, 0.0)
    out[...] = jnp.max(h2.reshape(_OW, _K, 128), axis=1)


def kernel(xyz_proj, points_proj, xyz_sampled_proj, W1, b1, W2, b2):
    f32 = jnp.float32
    xyz4 = jnp.pad(xyz_proj.reshape(_B * _NP, 3), ((0, 8), (0, 13)))
    ptstab = jnp.pad(points_proj.reshape(_B * _NP, _C), ((0, 8), (0, 0)))
    xyzflat = jnp.pad(xyz_proj.reshape(_B * _NP, 3),
                      ((0, 0), (0, 1))).reshape(_B * _NP * 4)

    mesh = plsc.VectorSubcoreMesh(core_axis_name="c", subcore_axis_name="s")
    fpts, fxyz = pl.kernel(
        _sc_body,
        out_type=[jax.ShapeDtypeStruct((_M, _C), f32),
                  jax.ShapeDtypeStruct((_M, 16), f32)],
        mesh=mesh,
        scratch_types=[
            pltpu.VMEM((6 * _W * 4,), f32),
            pltpu.VMEM((4, 1024), jnp.int32),
            pltpu.VMEM((1024, _C), f32),
            pltpu.VMEM((1024, 16), f32),
            pltpu.SemaphoreType.DMA,
            pltpu.SemaphoreType.DMA,
        ],
        compiler_params=pltpu.CompilerParams(needs_layout_passes=False,
                                             use_tc_tiling_on_sc=False),
    )(xyzflat, xyz4, ptstab)

    return _tc_part(fpts, fxyz, xyz_sampled_proj, W1, b1, W2, b2)


def _tc_part(fpts, fxyz, xyz_sampled_proj, W1, b1, W2, b2):
    f32 = jnp.float32
    nxr = jnp.broadcast_to(
        xyz_sampled_proj.reshape(_B, _N, 1, 3),
        (_B, _N, _K, 3)).reshape(_M, 3)
    grid = (_M // (_OW * _K),)
    out = pl.pallas_call(
        _tc_body,
        grid=grid,
        in_specs=[
            pl.BlockSpec((_OW * _K, _C), lambda m: (m, 0)),
            pl.BlockSpec((_OW * _K, 16), lambda m: (m, 0)),
            pl.BlockSpec((_OW * _K, 3), lambda m: (m, 0)),
            pl.BlockSpec((3, 64), lambda m: (0, 0)),
            pl.BlockSpec((64, 64), lambda m: (0, 0)),
            pl.BlockSpec((1, 64), lambda m: (0, 0)),
            pl.BlockSpec((64, 128), lambda m: (0, 0)),
            pl.BlockSpec((1, 128), lambda m: (0, 0)),
        ],
        out_specs=pl.BlockSpec((_OW, 128), lambda m: (m, 0)),
        out_shape=jax.ShapeDtypeStruct((_B * _N, 128), f32),
        compiler_params=pltpu.CompilerParams(
            dimension_semantics=("arbitrary",)),
    )(fpts, fxyz, nxr, W1[0:3], W1[3:], b1.reshape(1, 64), W2,
      b2.reshape(1, 128))
    return (out.reshape(_B, _N, 128), out.reshape(_B, _OH, _OW, 128))


# speedup vs baseline: 1.0111x; 1.0010x over previous
import functools
import jax
import jax.numpy as jnp
from jax import lax
from jax.experimental import pallas as pl
from jax.experimental.pallas import tpu as pltpu
from jax.experimental.pallas import tpu_sc as plsc

_B, _H, _W, _C = 2, 64, 512, 64
_OH, _OW = 32, 256
_K = 16
_R2 = 2.5 * 2.5
_N = _OH * _OW
_M = _B * _N * _K
_NP = _H * _W
_ZROW = _B * _NP
_NT = _ZROW + 8
_PAIRS = _B * _OH
_NW = 32
_PPW = _PAIRS // _NW


def _sc_body(xyzflat, xyztab, ptstab, fpts, fxyz,
             stage, slots, pbuf, xbuf, sem, sem2):
    i32 = jnp.int32
    wid = lax.axis_index("s") * 2 + lax.axis_index("c")
    lane = lax.iota(i32, 16)

    def do_pair(k, _):
        pair = wid * _PPW + k
        b = pair // _OH
        oh = pair % _OH
        s = jnp.clip(2 * oh - 3, 0, _H - 6)
        pltpu.sync_copy(
            xyzflat.at[pl.ds((b * _NP + s * _W) * 4, 6 * _W * 4)], stage)

        def init(t, _):
            slots[pl.ds(t * 16, 16)] = jnp.full((16,), _ZROW, i32)
            return _
        lax.fori_loop(0, 256, init, 0)

        def do_cg(cg, _):
            ow = cg * 16 + lane
            ow2 = 2 * ow
            slotbase = ow * _K
            cidx = ((2 * oh - s) * _W + ow2) * 4
            cx = plsc.load_gather(stage, [cidx])
            cy = plsc.load_gather(stage, [cidx + 1])
            cz = plsc.load_gather(stage, [cidx + 2])

            def row_cond(state):
                ih, cnt = state
                return jnp.logical_and(ih < 6, jnp.min(cnt) < _K)

            def row_body(state):
                ih, cnt = state
                row = 2 * oh + ih - 3
                rowok = jnp.logical_and(row >= 0, row < _H)
                rb = jnp.clip(row - s, 0, 5)
                rowbase = rb * (_W * 4)
                gbase = b * _NP + row * _W
                for iw in range(12):
                    col = ow2 + (iw - 6)
                    colok = jnp.logical_and(col >= 0, col < _W)
                    colc = jnp.clip(col, 0, _W - 1)
                    gi = rowbase + colc * 4
                    gx = plsc.load_gather(stage, [gi])
                    gy = plsc.load_gather(stage, [gi + 1])
                    gz = plsc.load_gather(stage, [gi + 2])
                    d2 = ((gx - cx) * (gx - cx) + (gy - cy) * (gy - cy)
                          + (gz - cz) * (gz - cz))
                    valid = jnp.logical_and(
                        jnp.logical_and(colok, d2 < _R2), rowok)
                    sel = jnp.logical_and(valid, cnt < _K)
                    plsc.store_scatter(slots, [slotbase + cnt],
                                       gbase + colc, mask=sel)
                    cnt = cnt + valid.astype(i32)
                return ih + 1, cnt

            lax.while_loop(row_cond, row_body,
                           (jnp.int32(0), jnp.zeros((16,), i32)))
            return _
        lax.fori_loop(0, 16, do_cg, 0)

        def do_chunk(ci, _):
            r0 = pair * (_OW * _K) + ci * 1024
            idxrow = slots.at[pl.ds(ci * 1024, 1024)]
            cp = pltpu.async_copy(ptstab.at[idxrow], pbuf, sem)
            cx2 = pltpu.async_copy(xyztab.at[idxrow], xbuf, sem2)
            cp.wait()
            cx2.wait()
            pltpu.sync_copy(pbuf, fpts.at[pl.ds(r0, 1024)])
            pltpu.sync_copy(xbuf, fxyz.at[pl.ds(r0, 1024)])
            return _
        lax.fori_loop(0, 4, do_chunk, 0)
        return _

    lax.fori_loop(0, _PPW, do_pair, 0)


def _tc_body(fpts, fxyz, nxr, w1x, w1p, b1, w2, b2, out):
    f32 = jnp.float32
    pts = fpts[...]
    xd = fxyz[...][:, 0:3] - nxr[...]
    h1 = (jnp.dot(pts.astype(jnp.bfloat16), w1p[...].astype(jnp.bfloat16),
                  preferred_element_type=f32)
          + jnp.dot(xd, w1x[...], preferred_element_type=f32) + b1[...])
    h1 = jnp.maximum(h1, 0.0)
    h2 = jnp.dot(h1.astype(jnp.bfloat16), w2[...].astype(jnp.bfloat16),
                 preferred_element_type=f32) + b2[...]
    h2 = jnp.maximum(h2, 0.0)
    out[...] = jnp.max(h2.reshape(_OW, _K, 128), axis=1)


def kernel(xyz_proj, points_proj, xyz_sampled_proj, W1, b1, W2, b2):
    f32 = jnp.float32
    xyz4 = jnp.pad(xyz_proj.reshape(_B * _NP, 3), ((0, 8), (0, 13)))
    ptstab = jnp.pad(points_proj.reshape(_B * _NP, _C), ((0, 8), (0, 0)))
    xyzflat = jnp.pad(xyz_proj.reshape(_B * _NP, 3),
                      ((0, 0), (0, 1))).reshape(_B * _NP * 4)

    mesh = plsc.VectorSubcoreMesh(core_axis_name="c", subcore_axis_name="s")
    fpts, fxyz = pl.kernel(
        _sc_body,
        out_type=[jax.ShapeDtypeStruct((_M, _C), f32),
                  jax.ShapeDtypeStruct((_M, 16), f32)],
        mesh=mesh,
        scratch_types=[
            pltpu.VMEM((6 * _W * 4,), f32),
            pltpu.VMEM((4096,), jnp.int32),
            pltpu.VMEM((1024, _C), f32),
            pltpu.VMEM((1024, 16), f32),
            pltpu.SemaphoreType.DMA,
            pltpu.SemaphoreType.DMA,
        ],
        compiler_params=pltpu.CompilerParams(needs_layout_passes=False,
                                             use_tc_tiling_on_sc=False),
    )(xyzflat, xyz4, ptstab)

    return _tc_part(fpts, fxyz, xyz_sampled_proj, W1, b1, W2, b2)


def _tc_part(fpts, fxyz, xyz_sampled_proj, W1, b1, W2, b2):
    f32 = jnp.float32
    nxr = jnp.broadcast_to(
        xyz_sampled_proj.reshape(_B, _N, 1, 3),
        (_B, _N, _K, 3)).reshape(_M, 3)
    grid = (_M // (_OW * _K),)
    out = pl.pallas_call(
        _tc_body,
        grid=grid,
        in_specs=[
            pl.BlockSpec((_OW * _K, _C), lambda m: (m, 0)),
            pl.BlockSpec((_OW * _K, 16), lambda m: (m, 0)),
            pl.BlockSpec((_OW * _K, 3), lambda m: (m, 0)),
            pl.BlockSpec((3, 64), lambda m: (0, 0)),
            pl.BlockSpec((64, 64), lambda m: (0, 0)),
            pl.BlockSpec((1, 64), lambda m: (0, 0)),
            pl.BlockSpec((64, 128), lambda m: (0, 0)),
            pl.BlockSpec((1, 128), lambda m: (0, 0)),
        ],
        out_specs=pl.BlockSpec((_OW, 128), lambda m: (m, 0)),
        out_shape=jax.ShapeDtypeStruct((_B * _N, 128), f32),
        compiler_params=pltpu.CompilerParams(
            dimension_semantics=("arbitrary",)),
    )(fpts, fxyz, nxr, W1[0:3], W1[3:], b1.reshape(1, 64), W2,
      b2.reshape(1, 128))
    return (out.reshape(_B, _N, 128), out.reshape(_B, _OH, _OW, 128))


# speedup vs baseline: 1.0223x; 1.0112x over previous
import functools
import jax
import jax.numpy as jnp
from jax import lax
from jax.experimental import pallas as pl
from jax.experimental.pallas import tpu as pltpu
from jax.experimental.pallas import tpu_sc as plsc

_B, _H, _W, _C = 2, 64, 512, 64
_OH, _OW = 32, 256
_K = 16
_R2 = 2.5 * 2.5
_N = _OH * _OW
_M = _B * _N * _K
_NP = _H * _W
_ZROW = _B * _NP
_NT = _ZROW + 8
_PAIRS = _B * _OH
_NW = 32
_PPW = _PAIRS // _NW


def _sc_body(xyzflat, xyztab, ptstab, fpts, fxyz,
             stage, slots, pbuf, xbuf, sem, sem2):
    i32 = jnp.int32
    wid = lax.axis_index("s") * 2 + lax.axis_index("c")
    lane = lax.iota(i32, 16)

    def do_pair(k, _):
        pair = wid * _PPW + k
        b = pair // _OH
        oh = pair % _OH
        s = jnp.clip(2 * oh - 3, 0, _H - 6)
        pltpu.sync_copy(
            xyzflat.at[pl.ds((b * _NP + s * _W) * 4, 6 * _W * 4)], stage)

        def init(t, _):
            slots[pl.ds(t * 16, 16)] = jnp.full((16,), _ZROW, i32)
            return _
        lax.fori_loop(0, 256, init, 0)

        def do_cg(cg, _):
            ow = cg * 16 + lane
            ow2 = 2 * ow
            slotbase = ow * _K
            cidx = ((2 * oh - s) * _W + ow2) * 4
            cx = plsc.load_gather(stage, [cidx])
            cy = plsc.load_gather(stage, [cidx + 1])
            cz = plsc.load_gather(stage, [cidx + 2])

            def row_cond(state):
                ih, cnt = state
                return jnp.logical_and(ih < 6, jnp.min(cnt) < _K)

            def row_body(state):
                ih, cnt = state
                row = 2 * oh + ih - 3
                rowok = jnp.logical_and(row >= 0, row < _H)
                rb = jnp.clip(row - s, 0, 5)
                rowbase = rb * (_W * 4)
                gbase = b * _NP + row * _W
                for iw in range(12):
                    col = ow2 + (iw - 6)
                    colok = jnp.logical_and(col >= 0, col < _W)
                    colc = jnp.clip(col, 0, _W - 1)
                    gi = rowbase + colc * 4
                    gx = plsc.load_gather(stage, [gi])
                    gy = plsc.load_gather(stage, [gi + 1])
                    gz = plsc.load_gather(stage, [gi + 2])
                    d2 = ((gx - cx) * (gx - cx) + (gy - cy) * (gy - cy)
                          + (gz - cz) * (gz - cz))
                    valid = jnp.logical_and(
                        jnp.logical_and(colok, d2 < _R2), rowok)
                    sel = jnp.logical_and(valid, cnt < _K)
                    plsc.store_scatter(slots, [slotbase + cnt],
                                       gbase + colc, mask=sel)
                    cnt = cnt + valid.astype(i32)
                return ih + 1, cnt

            lax.while_loop(row_cond, row_body,
                           (jnp.int32(0), jnp.zeros((16,), i32)))
            return _
        lax.fori_loop(0, 16, do_cg, 0)

        def do_chunk(ci, _):
            r0 = pair * (_OW * _K) + ci * 1024
            idxrow = slots.at[pl.ds(ci * 1024, 1024)]
            cp = pltpu.async_copy(ptstab.at[idxrow], pbuf, sem)
            cx2 = pltpu.async_copy(xyztab.at[idxrow], xbuf, sem2)
            cp.wait()
            cx2.wait()
            pltpu.sync_copy(pbuf, fpts.at[pl.ds(r0, 1024)])
            pltpu.sync_copy(xbuf, fxyz.at[pl.ds(r0, 1024)])
            return _
        lax.fori_loop(0, 4, do_chunk, 0)
        return _

    lax.fori_loop(0, _PPW, do_pair, 0)


def _tc_body(fpts, fxyz, nxr, w1x, w1p, b1, w2, b2, out):
    f32 = jnp.float32
    pts = fpts[...]
    xd = fxyz[...][:, 0:3] - nxr[...]
    h1 = (jnp.dot(pts, w1p[...].astype(jnp.bfloat16),
                  preferred_element_type=f32)
          + jnp.dot(xd, w1x[...], preferred_element_type=f32) + b1[...])
    h1 = jnp.maximum(h1, 0.0)
    h2 = jnp.dot(h1.astype(jnp.bfloat16), w2[...].astype(jnp.bfloat16),
                 preferred_element_type=f32) + b2[...]
    h2 = jnp.maximum(h2, 0.0)
    out[...] = jnp.max(h2.reshape(_OW, _K, 128), axis=1)


def kernel(xyz_proj, points_proj, xyz_sampled_proj, W1, b1, W2, b2):
    f32 = jnp.float32
    xyz4 = jnp.pad(xyz_proj.reshape(_B * _NP, 3), ((0, 8), (0, 13)))
    ptstab = jnp.pad(points_proj.reshape(_B * _NP, _C),
                     ((0, 8), (0, 0))).astype(jnp.bfloat16)
    xyzflat = jnp.pad(xyz_proj.reshape(_B * _NP, 3),
                      ((0, 0), (0, 1))).reshape(_B * _NP * 4)

    mesh = plsc.VectorSubcoreMesh(core_axis_name="c", subcore_axis_name="s")
    fpts, fxyz = pl.kernel(
        _sc_body,
        out_type=[jax.ShapeDtypeStruct((_M, _C), jnp.bfloat16),
                  jax.ShapeDtypeStruct((_M, 16), f32)],
        mesh=mesh,
        scratch_types=[
            pltpu.VMEM((6 * _W * 4,), f32),
            pltpu.VMEM((4096,), jnp.int32),
            pltpu.VMEM((1024, _C), jnp.bfloat16),
            pltpu.VMEM((1024, 16), f32),
            pltpu.SemaphoreType.DMA,
            pltpu.SemaphoreType.DMA,
        ],
        compiler_params=pltpu.CompilerParams(needs_layout_passes=False,
                                             use_tc_tiling_on_sc=False),
    )(xyzflat, xyz4, ptstab)

    return _tc_part(fpts, fxyz, xyz_sampled_proj, W1, b1, W2, b2)


def _tc_part(fpts, fxyz, xyz_sampled_proj, W1, b1, W2, b2):
    f32 = jnp.float32
    nxr = jnp.broadcast_to(
        xyz_sampled_proj.reshape(_B, _N, 1, 3),
        (_B, _N, _K, 3)).reshape(_M, 3)
    grid = (_M // (_OW * _K),)
    out = pl.pallas_call(
        _tc_body,
        grid=grid,
        in_specs=[
            pl.BlockSpec((_OW * _K, _C), lambda m: (m, 0)),
            pl.BlockSpec((_OW * _K, 16), lambda m: (m, 0)),
            pl.BlockSpec((_OW * _K, 3), lambda m: (m, 0)),
            pl.BlockSpec((3, 64), lambda m: (0, 0)),
            pl.BlockSpec((64, 64), lambda m: (0, 0)),
            pl.BlockSpec((1, 64), lambda m: (0, 0)),
            pl.BlockSpec((64, 128), lambda m: (0, 0)),
            pl.BlockSpec((1, 128), lambda m: (0, 0)),
        ],
        out_specs=pl.BlockSpec((_OW, 128), lambda m: (m, 0)),
        out_shape=jax.ShapeDtypeStruct((_B * _N, 128), f32),
        compiler_params=pltpu.CompilerParams(
            dimension_semantics=("arbitrary",)),
    )(fpts, fxyz, nxr, W1[0:3], W1[3:], b1.reshape(1, 64), W2,
      b2.reshape(1, 128))
    return (out.reshape(_B, _N, 128), out.reshape(_B, _OH, _OW, 128))


# speedup vs baseline: 1.0228x; 1.0004x over previous
import jax
import jax.numpy as jnp
from jax import lax
from jax.experimental import pallas as pl
from jax.experimental.pallas import tpu as pltpu
from jax.experimental.pallas import tpu_sc as plsc

_B, _H, _W, _C = 2, 64, 512, 64
_OH, _OW = 32, 256
_K = 16
_R2 = 2.5 * 2.5
_N = _OH * _OW
_M = _B * _N * _K
_NP = _H * _W
_ZROW = _B * _NP
_NT = _ZROW + 8
_PAIRS = _B * _OH
_NW = 32
_PPW = _PAIRS // _NW


def _sc_body(xyzflat, xyztab, ptstab, fpts, fxyz,
             stage, slots, pbuf, xbuf, sem, sem2):
    i32 = jnp.int32
    wid = lax.axis_index("s") * 2 + lax.axis_index("c")
    lane = lax.iota(i32, 16)

    def do_pair(k, _):
        pair = wid * _PPW + k
        b = pair // _OH
        oh = pair % _OH
        s = jnp.clip(2 * oh - 3, 0, _H - 6)
        pltpu.sync_copy(
            xyzflat.at[pl.ds((b * _NP + s * _W) * 4, 6 * _W * 4)], stage)

        def init(t, _):
            slots[pl.ds(t * 16, 16)] = jnp.full((16,), _ZROW, i32)
            return _
        lax.fori_loop(0, 256, init, 0)

        def do_cg(cg, _):
            ow = cg * 16 + lane
            ow2 = 2 * ow
            slotbase = ow * _K
            cidx = ((2 * oh - s) * _W + ow2) * 4
            cx = plsc.load_gather(stage, [cidx])
            cy = plsc.load_gather(stage, [cidx + 1])
            cz = plsc.load_gather(stage, [cidx + 2])

            def row_cond(state):
                ih, cnt = state
                return jnp.logical_and(ih < 6, jnp.min(cnt) < _K)

            def row_body(state):
                ih, cnt = state
                row = 2 * oh + ih - 3
                rowok = jnp.logical_and(row >= 0, row < _H)
                rb = jnp.clip(row - s, 0, 5)
                rowbase = rb * (_W * 4)
                gbase = b * _NP + row * _W
                for iw in range(12):
                    col = ow2 + (iw - 6)
                    colok = jnp.logical_and(col >= 0, col < _W)
                    colc = jnp.clip(col, 0, _W - 1)
                    gi = rowbase + colc * 4
                    gx = plsc.load_gather(stage, [gi])
                    gy = plsc.load_gather(stage, [gi + 1])
                    gz = plsc.load_gather(stage, [gi + 2])
                    d2 = ((gx - cx) * (gx - cx) + (gy - cy) * (gy - cy)
                          + (gz - cz) * (gz - cz))
                    valid = jnp.logical_and(
                        jnp.logical_and(colok, d2 < _R2), rowok)
                    sel = jnp.logical_and(valid, cnt < _K)
                    plsc.store_scatter(slots, [slotbase + cnt],
                                       gbase + colc, mask=sel)
                    cnt = cnt + valid.astype(i32)
                return ih + 1, cnt

            lax.while_loop(row_cond, row_body,
                           (jnp.int32(0), jnp.zeros((16,), i32)))
            return _
        lax.fori_loop(0, 16, do_cg, 0)

        def do_chunk(ci, _):
            r0 = pair * (_OW * _K) + ci * 1024
            idxrow = slots.at[pl.ds(ci * 1024, 1024)]
            cp = pltpu.async_copy(ptstab.at[idxrow], pbuf, sem)
            cx2 = pltpu.async_copy(xyztab.at[idxrow], xbuf, sem2)
            cp.wait()
            cx2.wait()
            pltpu.sync_copy(pbuf, fpts.at[pl.ds(r0, 1024)])
            pltpu.sync_copy(xbuf, fxyz.at[pl.ds(r0, 1024)])
            return _
        lax.fori_loop(0, 4, do_chunk, 0)
        return _

    lax.fori_loop(0, _PPW, do_pair, 0)


def _tc_body(fpts, fxyz, nxr, w1x, w1p, b1, w2, b2, out):
    f32 = jnp.float32
    pts = fpts[...]
    xd = fxyz[...][:, 0:3] - nxr[...]
    h1 = (jnp.dot(pts, w1p[...].astype(jnp.bfloat16),
                  preferred_element_type=f32)
          + jnp.dot(xd, w1x[...], preferred_element_type=f32) + b1[...])
    h1 = jnp.maximum(h1, 0.0)
    h2 = jnp.dot(h1.astype(jnp.bfloat16), w2[...].astype(jnp.bfloat16),
                 preferred_element_type=f32) + b2[...]
    h2 = jnp.maximum(h2, 0.0)
    out[...] = jnp.max(h2.reshape(_OW, _K, 128), axis=1)


def kernel(xyz_proj, points_proj, xyz_sampled_proj, W1, b1, W2, b2):
    f32 = jnp.float32
    xyz4 = jnp.pad(xyz_proj.reshape(_B * _NP, 3), ((0, 8), (0, 13)))
    ptstab = jnp.pad(points_proj.reshape(_B * _NP, _C),
                     ((0, 8), (0, 0))).astype(jnp.bfloat16)
    xyzflat = jnp.pad(xyz_proj.reshape(_B * _NP, 3),
                      ((0, 0), (0, 1))).reshape(_B * _NP * 4)

    mesh = plsc.VectorSubcoreMesh(core_axis_name="c", subcore_axis_name="s")
    fpts, fxyz = pl.kernel(
        _sc_body,
        out_type=[jax.ShapeDtypeStruct((_M, _C), jnp.bfloat16),
                  jax.ShapeDtypeStruct((_M, 16), f32)],
        mesh=mesh,
        scratch_types=[
            pltpu.VMEM((6 * _W * 4,), f32),
            pltpu.VMEM((4096,), jnp.int32),
            pltpu.VMEM((1024, _C), jnp.bfloat16),
            pltpu.VMEM((1024, 16), f32),
            pltpu.SemaphoreType.DMA,
            pltpu.SemaphoreType.DMA,
        ],
        compiler_params=pltpu.CompilerParams(needs_layout_passes=False,
                                             use_tc_tiling_on_sc=False),
    )(xyzflat, xyz4, ptstab)

    return _tc_part(fpts, fxyz, xyz_sampled_proj, W1, b1, W2, b2)


def _tc_part(fpts, fxyz, xyz_sampled_proj, W1, b1, W2, b2):
    f32 = jnp.float32
    nxr = jnp.broadcast_to(
        xyz_sampled_proj.reshape(_B, _N, 1, 3),
        (_B, _N, _K, 3)).reshape(_M, 3)
    grid = (_M // (_OW * _K),)
    out = pl.pallas_call(
        _tc_body,
        grid=grid,
        in_specs=[
            pl.BlockSpec((_OW * _K, _C), lambda m: (m, 0)),
            pl.BlockSpec((_OW * _K, 16), lambda m: (m, 0)),
            pl.BlockSpec((_OW * _K, 3), lambda m: (m, 0)),
            pl.BlockSpec((3, 64), lambda m: (0, 0)),
            pl.BlockSpec((64, 64), lambda m: (0, 0)),
            pl.BlockSpec((1, 64), lambda m: (0, 0)),
            pl.BlockSpec((64, 128), lambda m: (0, 0)),
            pl.BlockSpec((1, 128), lambda m: (0, 0)),
        ],
        out_specs=pl.BlockSpec((_OW, 128), lambda m: (m, 0)),
        out_shape=jax.ShapeDtypeStruct((_B * _N, 128), f32),
        compiler_params=pltpu.CompilerParams(
            dimension_semantics=("arbitrary",)),
    )(fpts, fxyz, nxr, W1[0:3], W1[3:], b1.reshape(1, 64), W2,
      b2.reshape(1, 128))
    return (out.reshape(_B, _N, 128), out.reshape(_B, _OH, _OW, 128))
